# baseline (speedup 1.0000x reference)
import functools

import jax
import jax.numpy as jnp
from jax import lax
from jax.experimental import pallas as pl
from jax.experimental.pallas import tpu as pltpu
from jax.experimental.pallas import tpu_sc as plsc

DT = 15
B, H, W = 4, 512, 512
N = B * DT * H * W

_NC, _NS = 2, 16
_NW = _NC * _NS
_CHUNK = N // _NW
_TILE = 49152
_NDMA = _CHUNK // _TILE


def _diff_body(x_ref, y_ref, m_ref, o_ref):
    d = y_ref[...] - x_ref[...]
    o_ref[...] = jnp.where(m_ref[...] != 0, d, 0.0)


_sc_mesh = plsc.VectorSubcoreMesh(core_axis_name="c", subcore_axis_name="s")


@functools.partial(
    pl.kernel,
    mesh=_sc_mesh,
    out_type=[
        jax.ShapeDtypeStruct((N,), jnp.float32),
        jax.ShapeDtypeStruct((N,), jnp.float32),
    ],
    scratch_types=[
        pltpu.VMEM((_TILE,), jnp.float32),
        pltpu.SemaphoreType.DMA,
    ],
)
def _sc_zeros(z0_hbm, z1_hbm, buf, sem):
    wid = lax.axis_index("s") * _NC + lax.axis_index("c")
    base = wid * _CHUNK

    def _fill(i, _):
        buf[pl.ds(i * 16, 16)] = jnp.zeros((16,), jnp.float32)
        return 0

    lax.fori_loop(0, _TILE // 16, _fill, 0)

    copies = []
    for out in (z0_hbm, z1_hbm):
        for i in range(_NDMA):
            copies.append(
                pltpu.async_copy(buf, out.at[pl.ds(base + i * _TILE, _TILE)], sem)
            )
    for c in copies:
        c.wait()


def kernel(x, ylr, msk_lr):
    m8 = msk_lr.view(jnp.int8)
    bt = 5
    grid = (B, DT // bt)
    out = pl.pallas_call(
        _diff_body,
        grid=grid,
        in_specs=[
            pl.BlockSpec((1, bt, H, W), lambda b, t: (b, t, 0, 0)),
            pl.BlockSpec((1, bt, H, W), lambda b, t: (b, t, 0, 0)),
            pl.BlockSpec((1, bt, H, W), lambda b, t: (b, t, 0, 0)),
        ],
        out_specs=pl.BlockSpec((1, bt, H, W), lambda b, t: (b, t, 0, 0)),
        out_shape=jax.ShapeDtypeStruct((B, DT, H, W), jnp.float32),
    )(x, ylr, m8)
    z0, z1 = _sc_zeros()
    shape = (B, DT, H, W)
    return out, z0.reshape(shape), z1.reshape(shape)


# speedup vs baseline: 1.0045x; 1.0045x over previous
import functools

import jax
import jax.numpy as jnp
from jax import lax
from jax.experimental import pallas as pl
from jax.experimental.pallas import tpu as pltpu
from jax.experimental.pallas import tpu_sc as plsc

DT = 15
B, H, W = 4, 512, 512
N = B * DT * H * W

_NC, _NS = 2, 16
_NW = _NC * _NS
_CHUNK = N // _NW
_TILE = 98304
_NDMA = _CHUNK // _TILE


def _diff_body(x_ref, y_ref, m_ref, o_ref):
    d = y_ref[...] - x_ref[...]
    o_ref[...] = jnp.where(m_ref[...] != 0, d, 0.0)


_sc_mesh = plsc.VectorSubcoreMesh(core_axis_name="c", subcore_axis_name="s")


@functools.partial(
    pl.kernel,
    mesh=_sc_mesh,
    out_type=[
        jax.ShapeDtypeStruct((N,), jnp.float32),
        jax.ShapeDtypeStruct((N,), jnp.float32),
    ],
    scratch_types=[
        pltpu.VMEM((_TILE,), jnp.float32),
        pltpu.SemaphoreType.DMA,
    ],
)
def _sc_zeros(z0_hbm, z1_hbm, buf, sem):
    wid = lax.axis_index("s") * _NC + lax.axis_index("c")
    base = wid * _CHUNK

    zv = jnp.zeros((16,), jnp.float32)

    def _fill(i, _):
        for j in range(16):
            buf[pl.ds(i * 256 + j * 16, 16)] = zv
        return 0

    lax.fori_loop(0, _TILE // 256, _fill, 0)

    copies = []
    for out in (z0_hbm, z1_hbm):
        for i in range(_NDMA):
            copies.append(
                pltpu.async_copy(buf, out.at[pl.ds(base + i * _TILE, _TILE)], sem)
            )
    for c in copies:
        c.wait()


def kernel(x, ylr, msk_lr):
    z0, z1 = _sc_zeros()
    m8 = msk_lr.view(jnp.int8)
    bt = 5
    grid = (B, DT // bt)
    out = pl.pallas_call(
        _diff_body,
        grid=grid,
        in_specs=[
            pl.BlockSpec((1, bt, H, W), lambda b, t: (b, t, 0, 0)),
            pl.BlockSpec((1, bt, H, W), lambda b, t: (b, t, 0, 0)),
            pl.BlockSpec((1, bt, H, W), lambda b, t: (b, t, 0, 0)),
        ],
        out_specs=pl.BlockSpec((1, bt, H, W), lambda b, t: (b, t, 0, 0)),
        out_shape=jax.ShapeDtypeStruct((B, DT, H, W), jnp.float32),
    )(x, ylr, m8)
    shape = (B, DT, H, W)
    return out, z0.reshape(shape), z1.reshape(shape)


# speedup vs baseline: 1.6327x; 1.6253x over previous
import functools

import jax
import jax.numpy as jnp
from jax import lax
from jax.experimental import pallas as pl
from jax.experimental.pallas import tpu as pltpu
from jax.experimental.pallas import tpu_sc as plsc

DT = 15
B, H, W = 4, 512, 512

_NC, _NS = 2, 16
_NW = _NC * _NS
_ROWS = 64
_SLABS_PER_PLANE = H // _ROWS
_PLANES = B * DT
_SLABS = _PLANES * _SLABS_PER_PLANE
_PER_W = _SLABS // _NW


def _diff_body(x_ref, y_ref, m_ref, o_ref):
    d = y_ref[...] - x_ref[...]
    o_ref[...] = jnp.where(m_ref[...], d, 0.0)


_sc_mesh = plsc.VectorSubcoreMesh(core_axis_name="c", subcore_axis_name="s")


@functools.partial(
    pl.kernel,
    mesh=_sc_mesh,
    out_type=[
        jax.ShapeDtypeStruct((B, DT, H, W), jnp.float32),
        jax.ShapeDtypeStruct((B, DT, H, W), jnp.float32),
    ],
    scratch_types=[
        pltpu.VMEM((_ROWS, W), jnp.float32),
        pltpu.SemaphoreType.DMA,
    ],
)
def _sc_zeros(z0_hbm, z1_hbm, buf, sem):
    wid = lax.axis_index("s") * _NC + lax.axis_index("c")

    zv = jnp.zeros((16,), jnp.float32)

    def _fill(r, _):
        for j in range(W // 16):
            buf[r, pl.ds(j * 16, 16)] = zv
        return 0

    lax.fori_loop(0, _ROWS, _fill, 0)

    copies = []
    for out in (z0_hbm, z1_hbm):
        for k in range(_PER_W):
            s = wid * _PER_W + k
            b = s // (DT * _SLABS_PER_PLANE)
            t = (s // _SLABS_PER_PLANE) % DT
            r0 = (s % _SLABS_PER_PLANE) * _ROWS
            copies.append(
                pltpu.async_copy(buf, out.at[b, t, pl.ds(r0, _ROWS)], sem)
            )
    for c in copies:
        c.wait()


def kernel(x, ylr, msk_lr):
    z0, z1 = _sc_zeros()
    bt = 5
    grid = (B, DT // bt)
    out = pl.pallas_call(
        _diff_body,
        grid=grid,
        in_specs=[
            pl.BlockSpec((1, bt, H, W), lambda b, t: (b, t, 0, 0)),
            pl.BlockSpec((1, bt, H, W), lambda b, t: (b, t, 0, 0)),
            pl.BlockSpec((1, bt, H, W), lambda b, t: (b, t, 0, 0)),
        ],
        out_specs=pl.BlockSpec((1, bt, H, W), lambda b, t: (b, t, 0, 0)),
        out_shape=jax.ShapeDtypeStruct((B, DT, H, W), jnp.float32),
    )(x, ylr, msk_lr)
    return out, z0, z1


# speedup vs baseline: 2.5170x; 1.5416x over previous
import jax
import jax.numpy as jnp
from jax.experimental import pallas as pl
from jax.experimental.pallas import tpu as pltpu

DT = 15
B, H, W = 4, 512, 512


def _body(x_ref, y_ref, m_ref, o_ref, z0_ref, z1_ref):
    d = y_ref[...] - x_ref[...]
    o_ref[...] = jnp.where(m_ref[...] != 0, d, 0.0)
    z0_ref[...] = jnp.zeros_like(z0_ref)
    z1_ref[...] = jnp.zeros_like(z1_ref)


def kernel(x, ylr, msk_lr):
    m8 = msk_lr.astype(jnp.int8)
    bt = 3
    grid = (B, DT // bt)
    spec = pl.BlockSpec((1, bt, H, W), lambda b, t: (b, t, 0, 0))
    oshape = jax.ShapeDtypeStruct((B, DT, H, W), jnp.float32)
    out, z0, z1 = pl.pallas_call(
        _body,
        grid=grid,
        in_specs=[spec, spec, spec],
        out_specs=[spec, spec, spec],
        out_shape=[oshape, oshape, oshape],
        compiler_params=pltpu.CompilerParams(
            dimension_semantics=("arbitrary", "arbitrary"),
            allow_input_fusion=(False, False, True),
        ),
    )(x, ylr, m8)
    return out, z0, z1
